# baseline (speedup 1.0000x reference)
import functools

import jax
import jax.numpy as jnp
from jax import lax
from jax.experimental import pallas as pl
from jax.experimental.pallas import tpu as pltpu
from jax.experimental.pallas import tpu_sc as plsc

N = 16777216
NC = 2
NS = 16
L = 16
NW = NC * NS
PER_W = N // NW
CHUNK = 8192
NCHUNK = PER_W // CHUNK
NBUF = 4
LA = NBUF - 1

_mesh = plsc.VectorSubcoreMesh(core_axis_name="c", subcore_axis_name="s")


@functools.partial(
    pl.kernel,
    mesh=_mesh,
    out_type=jax.ShapeDtypeStruct((N,), jnp.float32),
    scratch_types=(
        [pltpu.VMEM((NBUF, CHUNK), jnp.float32)] * 3
        + [pltpu.SemaphoreType.DMA] * (3 * NBUF)
    ),
)
def _stable_zero_div(x_hbm, y_hbm, out_hbm, x_v, y_v, o_v, *sems):
    sgx = sems[0:NBUF]
    sgy = sems[NBUF:2 * NBUF]
    ssc = sems[2 * NBUF:3 * NBUF]
    wid = lax.axis_index("s") * NC + lax.axis_index("c")
    base = wid * PER_W

    def start_gather(ci, t):
        off = base + ci * CHUNK
        hx = pltpu.async_copy(x_hbm.at[pl.ds(off, CHUNK)], x_v.at[t], sgx[t])
        hy = pltpu.async_copy(y_hbm.at[pl.ds(off, CHUNK)], y_v.at[t], sgy[t])
        return hx, hy

    def start_scatter(ci, t):
        off = base + ci * CHUNK
        return pltpu.async_copy(o_v.at[t], out_hbm.at[pl.ds(off, CHUNK)], ssc[t])

    def compute(t):
        @plsc.parallel_loop(0, CHUNK, step=L, unroll=8)
        def vec_body(i):
            s = pl.ds(i, L)
            xv = x_v[t, s]
            yv = y_v[t, s]
            o_v[t, s] = jnp.where(yv == 0.0, 0.0, xv / yv)

    gat = [None] * NBUF
    sca = [None] * NBUF
    for ci in range(min(LA, NCHUNK)):
        gat[ci % NBUF] = start_gather(ci, ci % NBUF)
    for ci in range(NCHUNK):
        s = ci % NBUF
        hx, hy = gat[s]
        hx.wait()
        hy.wait()
        if sca[s] is not None:
            sca[s].wait()
        compute(s)
        sca[s] = start_scatter(ci, s)
        nc = ci + LA
        if nc < NCHUNK:
            t = nc % NBUF
            gat[t] = start_gather(nc, t)
    for s in range(NBUF):
        if sca[s] is not None:
            sca[s].wait()


def kernel(x, y):
    return _stable_zero_div(x, y)


# speedup vs baseline: 1.2248x; 1.2248x over previous
import functools

import jax
import jax.numpy as jnp
from jax import lax
from jax.experimental import pallas as pl
from jax.experimental.pallas import tpu as pltpu
from jax.experimental.pallas import tpu_sc as plsc

N = 16777216
LANES = 128
ROWS = N // LANES

NC = 2
NS = 16
L = 16
NW = NC * NS
CHUNK = 16384
SC_CHUNKS_PER_W = 10
PER_W = SC_CHUNKS_PER_W * CHUNK
M = NW * PER_W
M_ROWS = M // LANES

_mesh = plsc.VectorSubcoreMesh(core_axis_name="c", subcore_axis_name="s")


@functools.partial(
    pl.kernel,
    mesh=_mesh,
    out_type=jax.ShapeDtypeStruct((M,), jnp.float32),
    scratch_types=(
        [pltpu.VMEM((2, CHUNK), jnp.float32)] * 3
        + [pltpu.SemaphoreType.DMA] * 6
    ),
)
def _sc_div(x_hbm, y_hbm, out_hbm, x_v, y_v, o_v, *sems):
    sgx = sems[0:2]
    sgy = sems[2:4]
    ssc = sems[4:6]
    wid = lax.axis_index("s") * NC + lax.axis_index("c")
    base = wid * PER_W

    def start_gather(ci, t):
        off = base + ci * CHUNK
        hx = pltpu.async_copy(x_hbm.at[pl.ds(off, CHUNK)], x_v.at[t], sgx[t])
        hy = pltpu.async_copy(y_hbm.at[pl.ds(off, CHUNK)], y_v.at[t], sgy[t])
        return hx, hy

    def start_scatter(ci, t):
        off = base + ci * CHUNK
        return pltpu.async_copy(o_v.at[t], out_hbm.at[pl.ds(off, CHUNK)], ssc[t])

    def compute(t):
        @plsc.parallel_loop(0, CHUNK, step=L, unroll=8)
        def vec_body(i):
            s = pl.ds(i, L)
            xv = x_v[t, s]
            yv = y_v[t, s]
            o_v[t, s] = jnp.where(yv == 0.0, 0.0, xv / yv)

    gat = [None, None]
    sca = [None, None]
    gat[0] = start_gather(0, 0)
    for ci in range(SC_CHUNKS_PER_W):
        s = ci & 1
        t = 1 - s
        if ci + 1 < SC_CHUNKS_PER_W:
            gat[t] = start_gather(ci + 1, t)
        hx, hy = gat[s]
        hx.wait()
        hy.wait()
        if sca[s] is not None:
            sca[s].wait()
        compute(s)
        sca[s] = start_scatter(ci, s)
    sca[0].wait()
    sca[1].wait()


TC_BLOCK_ROWS = 2048
TC_ROWS = ROWS - M_ROWS
TC_GRID = TC_ROWS // TC_BLOCK_ROWS
TC_ROW0 = M_ROWS // TC_BLOCK_ROWS


def _tc_div_body(x_ref, y_ref, o_ref):
    xv = x_ref[...]
    yv = y_ref[...]
    o_ref[...] = jnp.where(yv == 0.0, 0.0, xv / yv)


_tc_div = pl.pallas_call(
    _tc_div_body,
    grid=(TC_GRID,),
    in_specs=[
        pl.BlockSpec((TC_BLOCK_ROWS, LANES), lambda i: (TC_ROW0 + i, 0)),
        pl.BlockSpec((TC_BLOCK_ROWS, LANES), lambda i: (TC_ROW0 + i, 0)),
    ],
    out_specs=pl.BlockSpec((TC_BLOCK_ROWS, LANES), lambda i: (TC_ROW0 + i, 0)),
    out_shape=jax.ShapeDtypeStruct((ROWS, LANES), jnp.float32),
)


def _tc_merge_body(sc_ref, _, o_ref):
    o_ref[...] = sc_ref[...]


_tc_merge = pl.pallas_call(
    _tc_merge_body,
    grid=(M_ROWS // TC_BLOCK_ROWS,),
    in_specs=[
        pl.BlockSpec((TC_BLOCK_ROWS, LANES), lambda i: (i, 0)),
        pl.BlockSpec((TC_BLOCK_ROWS, LANES), lambda i: (i, 0)),
    ],
    out_specs=pl.BlockSpec((TC_BLOCK_ROWS, LANES), lambda i: (i, 0)),
    out_shape=jax.ShapeDtypeStruct((ROWS, LANES), jnp.float32),
    input_output_aliases={1: 0},
)


def kernel(x, y):
    sc_out = _sc_div(x, y)
    x2 = x.reshape(ROWS, LANES)
    y2 = y.reshape(ROWS, LANES)
    tc_out = _tc_div(x2, y2)
    merged = _tc_merge(sc_out.reshape(M_ROWS, LANES), tc_out)
    return merged.reshape(N)


# speedup vs baseline: 1.3166x; 1.0750x over previous
import functools

import jax
import jax.numpy as jnp
from jax import lax
from jax.experimental import pallas as pl
from jax.experimental.pallas import tpu as pltpu
from jax.experimental.pallas import tpu_sc as plsc

N = 16777216
LANES = 128
ROWS = N // LANES

NC = 2
NS = 16
L = 16
NW = NC * NS
CHUNK = 16384
SC_CHUNKS_PER_W = 10
PER_W = SC_CHUNKS_PER_W * CHUNK
M = NW * PER_W
M_ROWS = M // LANES

_mesh = plsc.VectorSubcoreMesh(core_axis_name="c", subcore_axis_name="s")


@functools.partial(
    pl.kernel,
    mesh=_mesh,
    out_type=jax.ShapeDtypeStruct((M,), jnp.float32),
    scratch_types=(
        [pltpu.VMEM((2, CHUNK), jnp.float32)] * 3
        + [pltpu.SemaphoreType.DMA] * 6
    ),
)
def _sc_div(x_hbm, y_hbm, out_hbm, x_v, y_v, o_v, *sems):
    sgx = sems[0:2]
    sgy = sems[2:4]
    ssc = sems[4:6]
    wid = lax.axis_index("s") * NC + lax.axis_index("c")
    base = wid * PER_W

    def start_gather(ci, t):
        off = base + ci * CHUNK
        hx = pltpu.async_copy(x_hbm.at[pl.ds(off, CHUNK)], x_v.at[t], sgx[t])
        hy = pltpu.async_copy(y_hbm.at[pl.ds(off, CHUNK)], y_v.at[t], sgy[t])
        return hx, hy

    def start_scatter(ci, t):
        off = base + ci * CHUNK
        return pltpu.async_copy(o_v.at[t], out_hbm.at[pl.ds(off, CHUNK)], ssc[t])

    def compute(t):
        @plsc.parallel_loop(0, CHUNK, step=L, unroll=8)
        def vec_body(i):
            s = pl.ds(i, L)
            xv = x_v[t, s]
            yv = y_v[t, s]
            o_v[t, s] = jnp.where(yv == 0.0, 0.0, xv / yv)

    gat = [None, None]
    sca = [None, None]
    gat[0] = start_gather(0, 0)
    for ci in range(SC_CHUNKS_PER_W):
        s = ci & 1
        t = 1 - s
        if ci + 1 < SC_CHUNKS_PER_W:
            gat[t] = start_gather(ci + 1, t)
        hx, hy = gat[s]
        hx.wait()
        hy.wait()
        if sca[s] is not None:
            sca[s].wait()
        compute(s)
        sca[s] = start_scatter(ci, s)
    sca[0].wait()
    sca[1].wait()


TC_BLOCK_ROWS = 4096
TC_ROWS = ROWS - M_ROWS
TC_GRID = TC_ROWS // TC_BLOCK_ROWS
TC_ROW0 = M_ROWS // TC_BLOCK_ROWS


def _tc_div_body(x_ref, y_ref, o_ref):
    xv = x_ref[...]
    yv = y_ref[...]
    o_ref[...] = jnp.where(yv == 0.0, 0.0, xv / yv)


_tc_div = pl.pallas_call(
    _tc_div_body,
    grid=(TC_GRID,),
    in_specs=[
        pl.BlockSpec((TC_BLOCK_ROWS, LANES), lambda i: (TC_ROW0 + i, 0)),
        pl.BlockSpec((TC_BLOCK_ROWS, LANES), lambda i: (TC_ROW0 + i, 0)),
    ],
    out_specs=pl.BlockSpec((TC_BLOCK_ROWS, LANES), lambda i: (TC_ROW0 + i, 0)),
    out_shape=jax.ShapeDtypeStruct((ROWS, LANES), jnp.float32),
)


def _tc_merge_body(sc_ref, _, o_ref):
    o_ref[...] = sc_ref[...]


_tc_merge = pl.pallas_call(
    _tc_merge_body,
    grid=(M_ROWS // TC_BLOCK_ROWS,),
    in_specs=[
        pl.BlockSpec((TC_BLOCK_ROWS, LANES), lambda i: (i, 0)),
        pl.BlockSpec((TC_BLOCK_ROWS, LANES), lambda i: (i, 0)),
    ],
    out_specs=pl.BlockSpec((TC_BLOCK_ROWS, LANES), lambda i: (i, 0)),
    out_shape=jax.ShapeDtypeStruct((ROWS, LANES), jnp.float32),
    input_output_aliases={1: 0},
)


def kernel(x, y):
    sc_out = _sc_div(x, y)
    x2 = x.reshape(ROWS, LANES)
    y2 = y.reshape(ROWS, LANES)
    tc_out = _tc_div(x2, y2)
    merged = _tc_merge(sc_out.reshape(M_ROWS, LANES), tc_out)
    return merged.reshape(N)


# speedup vs baseline: 1.3292x; 1.0096x over previous
import functools

import jax
import jax.numpy as jnp
from jax import lax
from jax.experimental import pallas as pl
from jax.experimental.pallas import tpu as pltpu
from jax.experimental.pallas import tpu_sc as plsc

N = 16777216
LANES = 128
ROWS = N // LANES

NC = 2
NS = 16
L = 16
NW = NC * NS
CHUNK = 16384
SC_CHUNKS_PER_W = 10
PER_W = SC_CHUNKS_PER_W * CHUNK
M = NW * PER_W
M_ROWS = M // LANES

_mesh = plsc.VectorSubcoreMesh(core_axis_name="c", subcore_axis_name="s")


@functools.partial(
    pl.kernel,
    mesh=_mesh,
    out_type=jax.ShapeDtypeStruct((M,), jnp.float32),
    scratch_types=(
        [pltpu.VMEM((2, CHUNK), jnp.float32)] * 3
        + [pltpu.SemaphoreType.DMA] * 6
    ),
)
def _sc_div(x_hbm, y_hbm, out_hbm, x_v, y_v, o_v, *sems):
    sgx = sems[0:2]
    sgy = sems[2:4]
    ssc = sems[4:6]
    wid = lax.axis_index("s") * NC + lax.axis_index("c")
    base = wid * PER_W

    def start_gather(ci, t):
        off = base + ci * CHUNK
        hx = pltpu.async_copy(x_hbm.at[pl.ds(off, CHUNK)], x_v.at[t], sgx[t])
        hy = pltpu.async_copy(y_hbm.at[pl.ds(off, CHUNK)], y_v.at[t], sgy[t])
        return hx, hy

    def start_scatter(ci, t):
        off = base + ci * CHUNK
        return pltpu.async_copy(o_v.at[t], out_hbm.at[pl.ds(off, CHUNK)], ssc[t])

    def compute(t):
        @plsc.parallel_loop(0, CHUNK, step=L, unroll=8)
        def vec_body(i):
            s = pl.ds(i, L)
            xv = x_v[t, s]
            yv = y_v[t, s]
            o_v[t, s] = jnp.where(yv == 0.0, 0.0, xv / yv)

    gat = [None, None]
    sca = [None, None]
    gat[0] = start_gather(0, 0)
    for ci in range(SC_CHUNKS_PER_W):
        s = ci & 1
        t = 1 - s
        if ci + 1 < SC_CHUNKS_PER_W:
            gat[t] = start_gather(ci + 1, t)
        hx, hy = gat[s]
        hx.wait()
        hy.wait()
        if sca[s] is not None:
            sca[s].wait()
        compute(s)
        sca[s] = start_scatter(ci, s)
    sca[0].wait()
    sca[1].wait()


TC_BLOCK_ROWS = 8192
TC_ROWS = ROWS - M_ROWS
TC_GRID = TC_ROWS // TC_BLOCK_ROWS
TC_ROW0 = M_ROWS // TC_BLOCK_ROWS


def _tc_div_body(x_ref, y_ref, o_ref):
    xv = x_ref[...]
    yv = y_ref[...]
    o_ref[...] = jnp.where(yv == 0.0, 0.0, xv / yv)


_tc_div = pl.pallas_call(
    _tc_div_body,
    grid=(TC_GRID,),
    in_specs=[
        pl.BlockSpec((TC_BLOCK_ROWS, LANES), lambda i: (TC_ROW0 + i, 0)),
        pl.BlockSpec((TC_BLOCK_ROWS, LANES), lambda i: (TC_ROW0 + i, 0)),
    ],
    out_specs=pl.BlockSpec((TC_BLOCK_ROWS, LANES), lambda i: (TC_ROW0 + i, 0)),
    out_shape=jax.ShapeDtypeStruct((ROWS, LANES), jnp.float32),
)


def _tc_merge_body(sc_ref, _, o_ref):
    o_ref[...] = sc_ref[...]


_tc_merge = pl.pallas_call(
    _tc_merge_body,
    grid=(M_ROWS // TC_BLOCK_ROWS,),
    in_specs=[
        pl.BlockSpec((TC_BLOCK_ROWS, LANES), lambda i: (i, 0)),
        pl.BlockSpec((TC_BLOCK_ROWS, LANES), lambda i: (i, 0)),
    ],
    out_specs=pl.BlockSpec((TC_BLOCK_ROWS, LANES), lambda i: (i, 0)),
    out_shape=jax.ShapeDtypeStruct((ROWS, LANES), jnp.float32),
    input_output_aliases={1: 0},
)


def kernel(x, y):
    sc_out = _sc_div(x, y)
    x2 = x.reshape(ROWS, LANES)
    y2 = y.reshape(ROWS, LANES)
    tc_out = _tc_div(x2, y2)
    merged = _tc_merge(sc_out.reshape(M_ROWS, LANES), tc_out)
    return merged.reshape(N)
